# Initial kernel scaffold; baseline (speedup 1.0000x reference)
import jax
import jax.numpy as jnp
from jax.experimental import pallas as pl


def kernel(x, pe_table):
    raise NotImplementedError("write your pallas kernel here")


# speedup vs baseline: 3.1808x; 3.1808x over previous
import jax
import jax.numpy as jnp
from jax.experimental import pallas as pl

BL = 1024


def _add_kernel(x_ref, pe_ref, o_ref):
    o_ref[...] = x_ref[...] + pe_ref[...]


def kernel(x, pe_table):
    B, L, D = x.shape
    grid = (L // BL, B)
    return pl.pallas_call(
        _add_kernel,
        grid=grid,
        in_specs=[
            pl.BlockSpec((1, BL, D), lambda i, b: (b, i, 0)),
            pl.BlockSpec((BL, D), lambda i, b: (i, 0)),
        ],
        out_specs=pl.BlockSpec((1, BL, D), lambda i, b: (b, i, 0)),
        out_shape=jax.ShapeDtypeStruct((B, L, D), x.dtype),
    )(x, pe_table)


# speedup vs baseline: 3.3122x; 1.0413x over previous
import jax
import jax.numpy as jnp
from jax.experimental import pallas as pl

BL = 2048


def _add_kernel(x_ref, pe_ref, o_ref):
    o_ref[...] = x_ref[...] + pe_ref[...]


def kernel(x, pe_table):
    B, L, D = x.shape
    grid = (L // BL, B)
    return pl.pallas_call(
        _add_kernel,
        grid=grid,
        in_specs=[
            pl.BlockSpec((1, BL, D), lambda i, b: (b, i, 0)),
            pl.BlockSpec((BL, D), lambda i, b: (i, 0)),
        ],
        out_specs=pl.BlockSpec((1, BL, D), lambda i, b: (b, i, 0)),
        out_shape=jax.ShapeDtypeStruct((B, L, D), x.dtype),
    )(x, pe_table)
